# Initial kernel scaffold; baseline (speedup 1.0000x reference)
import jax
import jax.numpy as jnp
from jax.experimental import pallas as pl


def kernel(x, edge_index, W1l, b1l, W1r, W2l, b2l, W2r, Wfc, bfc):
    raise NotImplementedError("write your pallas kernel here")


# speedup vs baseline: 7.9997x; 7.9997x over previous
import functools

import jax
import jax.numpy as jnp
from jax import lax
from jax.experimental import pallas as pl
from jax.experimental.pallas import tpu as pltpu
from jax.experimental.pallas import tpu_sc as plsc

N = 10000
E = 320000
F = 128
NCLASS = 64

NC = 2
NS = 16
NW = NC * NS

K = 128
E_PAD = 327680
EW = E_PAD // NW
CHUNKS = EW // K

ZROWS = 640
NTAB = NS * ZROWS
SCH = 80
NSTG = ZROWS // SCH

QHIST = 4
QLEN = EW // QHIST

BLK = 2000
GRID = N // BLK


def _sc_agg1_body(srcR, dstR, table, zfeat, zcnt,
                  aggp, cntp, idx_s, idx_d, rows, cnt_local, idx_q, acc,
                  sem, sem_i):
    cid = lax.axis_index("c")
    sid = lax.axis_index("s")
    wid = cid * NS + sid

    stage = rows.at[pl.ds(0, SCH)]
    pltpu.sync_copy(zfeat, stage)
    for j in range(NSTG):
        pltpu.sync_copy(stage, acc.at[pl.ds(sid * ZROWS + j * SCH, SCH)])
    pltpu.sync_copy(zcnt, cnt_local)
    plsc.subcore_barrier()

    def chunk(c, carry):
        e0 = wid * EW + c * K
        ia = pltpu.async_copy(srcR.at[pl.ds(e0, K)], idx_s, sem_i)
        ib = pltpu.async_copy(dstR.at[pl.ds(e0, K)], idx_d, sem_i)
        ia.wait()
        ib.wait()
        pltpu.async_copy(table.at[idx_s], rows, sem).wait()
        pltpu.sync_copy(rows, acc.at[idx_d], add=True)
        return carry

    lax.fori_loop(0, CHUNKS, chunk, 0)

    ones16 = jnp.broadcast_to(jnp.float32(1.0), (16,))
    for q in range(QHIST):
        pltpu.sync_copy(dstR.at[pl.ds(wid * EW + q * QLEN, QLEN)], idx_q)
        for i in range(QLEN // 16):
            dvec = idx_q[pl.ds(i * 16, 16)]
            plsc.addupdate_scatter(cnt_local, [dvec], ones16)

    plsc.subcore_barrier()

    row0 = cid * NTAB + sid * ZROWS
    for j in range(NSTG):
        pltpu.sync_copy(acc.at[pl.ds(sid * ZROWS + j * SCH, SCH)], stage)
        pltpu.sync_copy(stage, aggp.at[pl.ds(row0 + j * SCH, SCH)])
    pltpu.sync_copy(cnt_local, cntp.at[pl.ds(wid * NTAB, NTAB)])


def _sc_agg2_body(srcR, dstR, table, zfeat,
                  aggp, idx_sa, idx_da, idx_sb, idx_db, rows_a, rows_b,
                  acc, sem_g, sem_sa, sem_sb, sem_i):
    cid = lax.axis_index("c")
    sid = lax.axis_index("s")
    wid = cid * NS + sid

    stage = rows_a.at[pl.ds(0, SCH)]
    pltpu.sync_copy(zfeat, stage)
    for j in range(NSTG):
        pltpu.sync_copy(stage, acc.at[pl.ds(sid * ZROWS + j * SCH, SCH)])
    plsc.subcore_barrier()

    def chunk(c, carry):
        e0 = wid * EW + (2 * c) * K
        e1 = e0 + K
        ia = pltpu.async_copy(srcR.at[pl.ds(e0, K)], idx_sa, sem_i)
        ib = pltpu.async_copy(dstR.at[pl.ds(e0, K)], idx_da, sem_i)
        ic = pltpu.async_copy(srcR.at[pl.ds(e1, K)], idx_sb, sem_i)
        idd = pltpu.async_copy(dstR.at[pl.ds(e1, K)], idx_db, sem_i)
        ia.wait()
        ib.wait()
        ic.wait()
        idd.wait()
        pltpu.async_copy(table.at[idx_sa], rows_a, sem_g).wait()
        sa = pltpu.async_copy(rows_a, acc.at[idx_da], sem_sa, add=True)
        pltpu.async_copy(table.at[idx_sb], rows_b, sem_g).wait()
        sb = pltpu.async_copy(rows_b, acc.at[idx_db], sem_sb, add=True)
        sa.wait()
        sb.wait()
        return carry

    lax.fori_loop(0, CHUNKS // 2, chunk, 0)
    plsc.subcore_barrier()

    row0 = cid * NTAB + sid * ZROWS
    for j in range(NSTG):
        pltpu.sync_copy(acc.at[pl.ds(sid * ZROWS + j * SCH, SCH)], stage)
        pltpu.sync_copy(stage, aggp.at[pl.ds(row0 + j * SCH, SCH)])


def _make_sc_agg1():
    mesh = plsc.VectorSubcoreMesh(core_axis_name="c", subcore_axis_name="s")
    return pl.kernel(
        _sc_agg1_body,
        out_type=(jax.ShapeDtypeStruct((NC * NTAB, F), jnp.float32),
                  jax.ShapeDtypeStruct((NW * NTAB,), jnp.float32)),
        mesh=mesh,
        scratch_types=[
            pltpu.VMEM((K,), jnp.int32),
            pltpu.VMEM((K,), jnp.int32),
            pltpu.VMEM((K, F), jnp.float32),
            pltpu.VMEM((NTAB,), jnp.float32),
            pltpu.VMEM((QLEN,), jnp.int32),
            pltpu.VMEM_SHARED((NTAB, F), jnp.float32),
            pltpu.SemaphoreType.DMA,
            pltpu.SemaphoreType.DMA,
        ],
        compiler_params=pltpu.CompilerParams(needs_layout_passes=False),
    )


def _make_sc_agg2():
    mesh = plsc.VectorSubcoreMesh(core_axis_name="c", subcore_axis_name="s")
    return pl.kernel(
        _sc_agg2_body,
        out_type=jax.ShapeDtypeStruct((NC * NTAB, F), jnp.float32),
        mesh=mesh,
        scratch_types=[
            pltpu.VMEM((K,), jnp.int32),
            pltpu.VMEM((K,), jnp.int32),
            pltpu.VMEM((K,), jnp.int32),
            pltpu.VMEM((K,), jnp.int32),
            pltpu.VMEM((K, F), jnp.float32),
            pltpu.VMEM((K, F), jnp.float32),
            pltpu.VMEM_SHARED((NTAB, F), jnp.float32),
            pltpu.SemaphoreType.DMA,
            pltpu.SemaphoreType.DMA,
            pltpu.SemaphoreType.DMA,
            pltpu.SemaphoreType.DMA,
        ],
        compiler_params=pltpu.CompilerParams(needs_layout_passes=False),
    )


def _dense1_body(aggp, cntp, x, wl, wr, b, out):
    agg = aggp[0] + aggp[1]
    cnt = jnp.sum(cntp[...], axis=1, keepdims=True)
    mean = agg / jnp.maximum(cnt, 1.0)
    t = (jnp.dot(mean, wl[...], preferred_element_type=jnp.float32,
                 precision=lax.Precision.HIGHEST)
         + jnp.dot(x[...], wr[...], preferred_element_type=jnp.float32,
                   precision=lax.Precision.HIGHEST)
         + b[...])
    h = jnp.maximum(t, 0.0)
    nrm = jnp.sqrt(jnp.sum(h * h, axis=1, keepdims=True))
    out[...] = h / jnp.maximum(nrm, 1e-12)


def _dense2_body(aggp, cntp, x, wl, wr, b, wfc, bfc, out):
    agg = aggp[0] + aggp[1]
    cnt = jnp.sum(cntp[...], axis=1, keepdims=True)
    mean = agg / jnp.maximum(cnt, 1.0)
    t = (jnp.dot(mean, wl[...], preferred_element_type=jnp.float32,
                 precision=lax.Precision.HIGHEST)
         + jnp.dot(x[...], wr[...], preferred_element_type=jnp.float32,
                   precision=lax.Precision.HIGHEST)
         + b[...])
    h = jnp.maximum(t, 0.0)
    nrm = jnp.sqrt(jnp.sum(h * h, axis=1, keepdims=True))
    hn = h / jnp.maximum(nrm, 1e-12)
    out[...] = jnp.dot(hn, wfc[...], preferred_element_type=jnp.float32,
                       precision=lax.Precision.HIGHEST) + bfc[...]


def _dense_common_specs():
    return [
        pl.BlockSpec((NC, BLK, F), lambda i: (0, i, 0)),
        pl.BlockSpec((BLK, NW), lambda i: (i, 0)),
        pl.BlockSpec((BLK, F), lambda i: (i, 0)),
        pl.BlockSpec((F, F), lambda i: (0, 0)),
        pl.BlockSpec((F, F), lambda i: (0, 0)),
        pl.BlockSpec((1, F), lambda i: (0, 0)),
    ]


def _dense1(aggp, cntp, x, wlT, wrT, b):
    return pl.pallas_call(
        _dense1_body,
        grid=(GRID,),
        in_specs=_dense_common_specs(),
        out_specs=pl.BlockSpec((BLK, F), lambda i: (i, 0)),
        out_shape=jax.ShapeDtypeStruct((N, F), jnp.float32),
    )(aggp, cntp, x, wlT, wrT, b)


def _dense2(aggp, cntp, x, wlT, wrT, b, wfcT, bfc):
    specs = _dense_common_specs() + [
        pl.BlockSpec((F, NCLASS), lambda i: (0, 0)),
        pl.BlockSpec((1, NCLASS), lambda i: (0, 0)),
    ]
    return pl.pallas_call(
        _dense2_body,
        grid=(GRID,),
        in_specs=specs,
        out_specs=pl.BlockSpec((BLK, NCLASS), lambda i: (i, 0)),
        out_shape=jax.ShapeDtypeStruct((N, NCLASS), jnp.float32),
    )(aggp, cntp, x, wlT, wrT, b, wfcT, bfc)


def kernel(x, edge_index, W1l, b1l, W1r, W2l, b2l, W2r, Wfc, bfc):
    ei = edge_index.astype(jnp.int32)
    pad = E_PAD - E
    ar = jnp.arange(pad, dtype=jnp.int32)
    src = jnp.concatenate([ei[0], ar % N])
    dst = jnp.concatenate([ei[1], N + ar % (NTAB - N)])

    zfeat = jnp.zeros((SCH, F), jnp.float32)
    zcnt = jnp.zeros((NTAB,), jnp.float32)

    aggp1, cntp = _make_sc_agg1()(src, dst, x, zfeat, zcnt)
    aggp1 = aggp1.reshape(NC, NTAB, F)
    cntp = cntp.reshape(NW, NTAB).T
    h = _dense1(aggp1, cntp, x, W1l.T, W1r.T, b1l.reshape(1, F))
    aggp2 = _make_sc_agg2()(src, dst, h, zfeat).reshape(NC, NTAB, F)
    out = _dense2(aggp2, cntp, h, W2l.T, W2r.T, b2l.reshape(1, F),
                  Wfc.T, bfc.reshape(1, NCLASS))
    return out


# speedup vs baseline: 8.1640x; 1.0205x over previous
import functools

import jax
import jax.numpy as jnp
from jax import lax
from jax.experimental import pallas as pl
from jax.experimental.pallas import tpu as pltpu
from jax.experimental.pallas import tpu_sc as plsc

N = 10000
E = 320000
F = 128
NCLASS = 64

NC = 2
NS = 16
NW = NC * NS

K = 128
E_PAD = 327680
EW = E_PAD // NW
CHUNKS = EW // K

ZROWS = 640
NTAB = NS * ZROWS
SCH = 64
NSTG = ZROWS // SCH

QHIST = 4
QLEN = EW // QHIST

BLK = 2000
GRID = N // BLK


def _sc_agg1_body(srcR, dstR, table, zfeat, zcnt,
                  aggp, cntp, idx_s, idx_d, rows, cnt_local, idx_q, acc,
                  sem, sem_i):
    cid = lax.axis_index("c")
    sid = lax.axis_index("s")
    wid = cid * NS + sid

    stage_a = rows.at[pl.ds(0, SCH)]
    stage_b = rows.at[pl.ds(SCH, SCH)]
    pltpu.sync_copy(zfeat, stage_a)
    zd = [pltpu.async_copy(stage_a, acc.at[pl.ds(sid * ZROWS + j * SCH, SCH)],
                           sem_i) for j in range(NSTG)]
    pltpu.sync_copy(zcnt, cnt_local)
    for d in zd:
        d.wait()
    plsc.subcore_barrier()

    def chunk(c, carry):
        e0 = wid * EW + c * K
        ia = pltpu.async_copy(srcR.at[pl.ds(e0, K)], idx_s, sem_i)
        ib = pltpu.async_copy(dstR.at[pl.ds(e0, K)], idx_d, sem_i)
        ia.wait()
        ib.wait()
        pltpu.async_copy(table.at[idx_s], rows, sem).wait()
        pltpu.sync_copy(rows, acc.at[idx_d], add=True)
        return carry

    lax.fori_loop(0, CHUNKS, chunk, 0)

    ones16 = jnp.broadcast_to(jnp.float32(1.0), (16,))
    for q in range(QHIST):
        pltpu.sync_copy(dstR.at[pl.ds(wid * EW + q * QLEN, QLEN)], idx_q)
        for i in range(QLEN // 16):
            dvec = idx_q[pl.ds(i * 16, 16)]
            plsc.addupdate_scatter(cnt_local, [dvec], ones16)

    plsc.subcore_barrier()

    row0 = cid * NTAB + sid * ZROWS
    wd = []
    for j in range(NSTG):
        st = stage_a if j % 2 == 0 else stage_b
        if j >= 2:
            wd[j - 2].wait()
        pltpu.sync_copy(acc.at[pl.ds(sid * ZROWS + j * SCH, SCH)], st)
        wd.append(pltpu.async_copy(st, aggp.at[pl.ds(row0 + j * SCH, SCH)],
                                   sem_i))
    pltpu.sync_copy(cnt_local, cntp.at[pl.ds(wid * NTAB, NTAB)])
    wd[-2].wait()
    wd[-1].wait()


def _sc_agg2_body(srcR, dstR, table, zfeat,
                  aggp, idx_sa, idx_da, idx_sb, idx_db, rows_a, rows_b,
                  acc, sem_g, sem_sa, sem_sb, sem_i):
    cid = lax.axis_index("c")
    sid = lax.axis_index("s")
    wid = cid * NS + sid

    stage_a = rows_a.at[pl.ds(0, SCH)]
    stage_b = rows_b.at[pl.ds(0, SCH)]
    pltpu.sync_copy(zfeat, stage_a)
    zd = [pltpu.async_copy(stage_a, acc.at[pl.ds(sid * ZROWS + j * SCH, SCH)],
                           sem_i) for j in range(NSTG)]
    for d in zd:
        d.wait()
    plsc.subcore_barrier()

    def chunk(c, carry):
        e0 = wid * EW + (2 * c) * K
        e1 = e0 + K
        ia = pltpu.async_copy(srcR.at[pl.ds(e0, K)], idx_sa, sem_i)
        ib = pltpu.async_copy(dstR.at[pl.ds(e0, K)], idx_da, sem_i)
        ic = pltpu.async_copy(srcR.at[pl.ds(e1, K)], idx_sb, sem_i)
        idd = pltpu.async_copy(dstR.at[pl.ds(e1, K)], idx_db, sem_i)
        ia.wait()
        ib.wait()
        ic.wait()
        idd.wait()
        pltpu.async_copy(table.at[idx_sa], rows_a, sem_g).wait()
        sa = pltpu.async_copy(rows_a, acc.at[idx_da], sem_sa, add=True)
        pltpu.async_copy(table.at[idx_sb], rows_b, sem_g).wait()
        sb = pltpu.async_copy(rows_b, acc.at[idx_db], sem_sb, add=True)
        sa.wait()
        sb.wait()
        return carry

    lax.fori_loop(0, CHUNKS // 2, chunk, 0)
    plsc.subcore_barrier()

    row0 = cid * NTAB + sid * ZROWS
    wd = []
    for j in range(NSTG):
        st = stage_a if j % 2 == 0 else stage_b
        if j >= 2:
            wd[j - 2].wait()
        pltpu.sync_copy(acc.at[pl.ds(sid * ZROWS + j * SCH, SCH)], st)
        wd.append(pltpu.async_copy(st, aggp.at[pl.ds(row0 + j * SCH, SCH)],
                                   sem_i))
    wd[-2].wait()
    wd[-1].wait()


def _make_sc_agg1():
    mesh = plsc.VectorSubcoreMesh(core_axis_name="c", subcore_axis_name="s")
    return pl.kernel(
        _sc_agg1_body,
        out_type=(jax.ShapeDtypeStruct((NC * NTAB, F), jnp.float32),
                  jax.ShapeDtypeStruct((NW * NTAB,), jnp.float32)),
        mesh=mesh,
        scratch_types=[
            pltpu.VMEM((K,), jnp.int32),
            pltpu.VMEM((K,), jnp.int32),
            pltpu.VMEM((K, F), jnp.float32),
            pltpu.VMEM((NTAB,), jnp.float32),
            pltpu.VMEM((QLEN,), jnp.int32),
            pltpu.VMEM_SHARED((NTAB, F), jnp.float32),
            pltpu.SemaphoreType.DMA,
            pltpu.SemaphoreType.DMA,
        ],
        compiler_params=pltpu.CompilerParams(needs_layout_passes=False),
    )


def _make_sc_agg2():
    mesh = plsc.VectorSubcoreMesh(core_axis_name="c", subcore_axis_name="s")
    return pl.kernel(
        _sc_agg2_body,
        out_type=jax.ShapeDtypeStruct((NC * NTAB, F), jnp.float32),
        mesh=mesh,
        scratch_types=[
            pltpu.VMEM((K,), jnp.int32),
            pltpu.VMEM((K,), jnp.int32),
            pltpu.VMEM((K,), jnp.int32),
            pltpu.VMEM((K,), jnp.int32),
            pltpu.VMEM((K, F), jnp.float32),
            pltpu.VMEM((K, F), jnp.float32),
            pltpu.VMEM_SHARED((NTAB, F), jnp.float32),
            pltpu.SemaphoreType.DMA,
            pltpu.SemaphoreType.DMA,
            pltpu.SemaphoreType.DMA,
            pltpu.SemaphoreType.DMA,
        ],
        compiler_params=pltpu.CompilerParams(needs_layout_passes=False),
    )


def _dense1_body(aggp, cntp, x, wl, wr, b, out):
    agg = aggp[0] + aggp[1]
    cnt = jnp.sum(cntp[...], axis=1, keepdims=True)
    mean = agg / jnp.maximum(cnt, 1.0)
    t = (jnp.dot(mean, wl[...], preferred_element_type=jnp.float32,
                 precision=lax.Precision.HIGHEST)
         + jnp.dot(x[...], wr[...], preferred_element_type=jnp.float32,
                   precision=lax.Precision.HIGHEST)
         + b[...])
    h = jnp.maximum(t, 0.0)
    nrm = jnp.sqrt(jnp.sum(h * h, axis=1, keepdims=True))
    out[...] = h / jnp.maximum(nrm, 1e-12)


def _dense2_body(aggp, cntp, x, wl, wr, b, wfc, bfc, out):
    agg = aggp[0] + aggp[1]
    cnt = jnp.sum(cntp[...], axis=1, keepdims=True)
    mean = agg / jnp.maximum(cnt, 1.0)
    t = (jnp.dot(mean, wl[...], preferred_element_type=jnp.float32,
                 precision=lax.Precision.HIGHEST)
         + jnp.dot(x[...], wr[...], preferred_element_type=jnp.float32,
                   precision=lax.Precision.HIGHEST)
         + b[...])
    h = jnp.maximum(t, 0.0)
    nrm = jnp.sqrt(jnp.sum(h * h, axis=1, keepdims=True))
    hn = h / jnp.maximum(nrm, 1e-12)
    out[...] = jnp.dot(hn, wfc[...], preferred_element_type=jnp.float32,
                       precision=lax.Precision.HIGHEST) + bfc[...]


def _dense_common_specs():
    return [
        pl.BlockSpec((NC, BLK, F), lambda i: (0, i, 0)),
        pl.BlockSpec((BLK, NW), lambda i: (i, 0)),
        pl.BlockSpec((BLK, F), lambda i: (i, 0)),
        pl.BlockSpec((F, F), lambda i: (0, 0)),
        pl.BlockSpec((F, F), lambda i: (0, 0)),
        pl.BlockSpec((1, F), lambda i: (0, 0)),
    ]


def _dense1(aggp, cntp, x, wlT, wrT, b):
    return pl.pallas_call(
        _dense1_body,
        grid=(GRID,),
        in_specs=_dense_common_specs(),
        out_specs=pl.BlockSpec((BLK, F), lambda i: (i, 0)),
        out_shape=jax.ShapeDtypeStruct((N, F), jnp.float32),
    )(aggp, cntp, x, wlT, wrT, b)


def _dense2(aggp, cntp, x, wlT, wrT, b, wfcT, bfc):
    specs = _dense_common_specs() + [
        pl.BlockSpec((F, NCLASS), lambda i: (0, 0)),
        pl.BlockSpec((1, NCLASS), lambda i: (0, 0)),
    ]
    return pl.pallas_call(
        _dense2_body,
        grid=(GRID,),
        in_specs=specs,
        out_specs=pl.BlockSpec((BLK, NCLASS), lambda i: (i, 0)),
        out_shape=jax.ShapeDtypeStruct((N, NCLASS), jnp.float32),
    )(aggp, cntp, x, wlT, wrT, b, wfcT, bfc)


def kernel(x, edge_index, W1l, b1l, W1r, W2l, b2l, W2r, Wfc, bfc):
    ei = edge_index.astype(jnp.int32)
    pad = E_PAD - E
    ar = jnp.arange(pad, dtype=jnp.int32)
    src = jnp.concatenate([ei[0], ar % N])
    dst = jnp.concatenate([ei[1], N + ar % (NTAB - N)])

    zfeat = jnp.zeros((SCH, F), jnp.float32)
    zcnt = jnp.zeros((NTAB,), jnp.float32)

    aggp1, cntp = _make_sc_agg1()(src, dst, x, zfeat, zcnt)
    aggp1 = aggp1.reshape(NC, NTAB, F)
    cntp = cntp.reshape(NW, NTAB).T
    h = _dense1(aggp1, cntp, x, W1l.T, W1r.T, b1l.reshape(1, F))
    aggp2 = _make_sc_agg2()(src, dst, h, zfeat).reshape(NC, NTAB, F)
    out = _dense2(aggp2, cntp, h, W2l.T, W2r.T, b2l.reshape(1, F),
                  Wfc.T, bfc.reshape(1, NCLASS))
    return out


# speedup vs baseline: 8.2192x; 1.0068x over previous
import functools

import jax
import jax.numpy as jnp
from jax import lax
from jax.experimental import pallas as pl
from jax.experimental.pallas import tpu as pltpu
from jax.experimental.pallas import tpu_sc as plsc

N = 10000
E = 320000
F = 128
NCLASS = 64

NC = 2
NS = 16
NW = NC * NS

K = 128
E_PAD = 327680
EW = E_PAD // NW
CHUNKS = EW // K

ZROWS = 640
NTAB = NS * ZROWS
SCH = 64
NSTG = ZROWS // SCH

QHIST = 4
QLEN = EW // QHIST

BLK = 2000
GRID = N // BLK


def _sc_agg1_body(srcR, dstR, table, zfeat, zcnt,
                  aggp, cntp, idx_s, idx_d, rows, cnt_local, idx_q, acc,
                  sem, sem_i):
    cid = lax.axis_index("c")
    sid = lax.axis_index("s")
    wid = cid * NS + sid

    stage_a = rows.at[pl.ds(0, SCH)]
    stage_b = rows.at[pl.ds(SCH, SCH)]
    pltpu.sync_copy(zfeat, stage_a)
    zd = [pltpu.async_copy(stage_a, acc.at[pl.ds(sid * ZROWS + j * SCH, SCH)],
                           sem_i) for j in range(NSTG)]
    pltpu.sync_copy(zcnt, cnt_local)
    for d in zd:
        d.wait()
    plsc.subcore_barrier()

    def chunk(c, carry):
        e0 = wid * EW + c * K
        ia = pltpu.async_copy(srcR.at[pl.ds(e0, K)], idx_s, sem_i)
        ib = pltpu.async_copy(dstR.at[pl.ds(e0, K)], idx_d, sem_i)
        ia.wait()
        ib.wait()
        pltpu.async_copy(table.at[idx_s], rows, sem).wait()
        pltpu.sync_copy(rows, acc.at[idx_d], add=True)
        return carry

    lax.fori_loop(0, CHUNKS, chunk, 0)

    ones16 = jnp.broadcast_to(jnp.float32(1.0), (16,))
    for q in range(QHIST):
        pltpu.sync_copy(dstR.at[pl.ds(wid * EW + q * QLEN, QLEN)], idx_q)
        for i in range(QLEN // 16):
            dvec = idx_q[pl.ds(i * 16, 16)]
            plsc.addupdate_scatter(cnt_local, [dvec], ones16)

    plsc.subcore_barrier()

    row0 = cid * NTAB + sid * ZROWS
    wd = []
    for j in range(NSTG):
        st = stage_a if j % 2 == 0 else stage_b
        if j >= 2:
            wd[j - 2].wait()
        pltpu.sync_copy(acc.at[pl.ds(sid * ZROWS + j * SCH, SCH)], st)
        wd.append(pltpu.async_copy(st, aggp.at[pl.ds(row0 + j * SCH, SCH)],
                                   sem_i))
    pltpu.sync_copy(cnt_local, cntp.at[pl.ds(wid * NTAB, NTAB)])
    wd[-2].wait()
    wd[-1].wait()


def _sc_agg2_body(srcR, dstR, table, zfeat,
                  aggp, idx_sa, idx_da, idx_sb, idx_db, rows_a, rows_b,
                  acc, sem_g, sem_sa, sem_sb, sem_i, sem_gb):
    cid = lax.axis_index("c")
    sid = lax.axis_index("s")
    wid = cid * NS + sid

    stage_a = rows_a.at[pl.ds(0, SCH)]
    stage_b = rows_b.at[pl.ds(0, SCH)]
    pltpu.sync_copy(zfeat, stage_a)
    zd = [pltpu.async_copy(stage_a, acc.at[pl.ds(sid * ZROWS + j * SCH, SCH)],
                           sem_i) for j in range(NSTG)]
    for d in zd:
        d.wait()
    plsc.subcore_barrier()

    def chunk(c, carry):
        e0 = wid * EW + (2 * c) * K
        e1 = e0 + K
        ia = pltpu.async_copy(srcR.at[pl.ds(e0, K)], idx_sa, sem_i)
        ib = pltpu.async_copy(dstR.at[pl.ds(e0, K)], idx_da, sem_i)
        ic = pltpu.async_copy(srcR.at[pl.ds(e1, K)], idx_sb, sem_i)
        idd = pltpu.async_copy(dstR.at[pl.ds(e1, K)], idx_db, sem_i)
        ia.wait()
        ib.wait()
        ic.wait()
        idd.wait()
        ga = pltpu.async_copy(table.at[idx_sa], rows_a, sem_g)
        gb = pltpu.async_copy(table.at[idx_sb], rows_b, sem_gb)
        ga.wait()
        sa = pltpu.async_copy(rows_a, acc.at[idx_da], sem_sa, add=True)
        gb.wait()
        sb = pltpu.async_copy(rows_b, acc.at[idx_db], sem_sb, add=True)
        sa.wait()
        sb.wait()
        return carry

    lax.fori_loop(0, CHUNKS // 2, chunk, 0)
    plsc.subcore_barrier()

    row0 = cid * NTAB + sid * ZROWS
    wd = []
    for j in range(NSTG):
        st = stage_a if j % 2 == 0 else stage_b
        if j >= 2:
            wd[j - 2].wait()
        pltpu.sync_copy(acc.at[pl.ds(sid * ZROWS + j * SCH, SCH)], st)
        wd.append(pltpu.async_copy(st, aggp.at[pl.ds(row0 + j * SCH, SCH)],
                                   sem_i))
    wd[-2].wait()
    wd[-1].wait()


def _make_sc_agg1():
    mesh = plsc.VectorSubcoreMesh(core_axis_name="c", subcore_axis_name="s")
    return pl.kernel(
        _sc_agg1_body,
        out_type=(jax.ShapeDtypeStruct((NC * NTAB, F), jnp.float32),
                  jax.ShapeDtypeStruct((NW * NTAB,), jnp.float32)),
        mesh=mesh,
        scratch_types=[
            pltpu.VMEM((K,), jnp.int32),
            pltpu.VMEM((K,), jnp.int32),
            pltpu.VMEM((K, F), jnp.float32),
            pltpu.VMEM((NTAB,), jnp.float32),
            pltpu.VMEM((QLEN,), jnp.int32),
            pltpu.VMEM_SHARED((NTAB, F), jnp.float32),
            pltpu.SemaphoreType.DMA,
            pltpu.SemaphoreType.DMA,
        ],
        compiler_params=pltpu.CompilerParams(needs_layout_passes=False),
    )


def _make_sc_agg2():
    mesh = plsc.VectorSubcoreMesh(core_axis_name="c", subcore_axis_name="s")
    return pl.kernel(
        _sc_agg2_body,
        out_type=jax.ShapeDtypeStruct((NC * NTAB, F), jnp.float32),
        mesh=mesh,
        scratch_types=[
            pltpu.VMEM((K,), jnp.int32),
            pltpu.VMEM((K,), jnp.int32),
            pltpu.VMEM((K,), jnp.int32),
            pltpu.VMEM((K,), jnp.int32),
            pltpu.VMEM((K, F), jnp.float32),
            pltpu.VMEM((K, F), jnp.float32),
            pltpu.VMEM_SHARED((NTAB, F), jnp.float32),
            pltpu.SemaphoreType.DMA,
            pltpu.SemaphoreType.DMA,
            pltpu.SemaphoreType.DMA,
            pltpu.SemaphoreType.DMA,
            pltpu.SemaphoreType.DMA,
        ],
        compiler_params=pltpu.CompilerParams(needs_layout_passes=False),
    )


def _dense1_body(aggp, cntp, x, wl, wr, b, out):
    agg = aggp[0] + aggp[1]
    cnt = jnp.sum(cntp[...], axis=1, keepdims=True)
    mean = agg / jnp.maximum(cnt, 1.0)
    t = (jnp.dot(mean, wl[...], preferred_element_type=jnp.float32,
                 precision=lax.Precision.HIGHEST)
         + jnp.dot(x[...], wr[...], preferred_element_type=jnp.float32,
                   precision=lax.Precision.HIGHEST)
         + b[...])
    h = jnp.maximum(t, 0.0)
    nrm = jnp.sqrt(jnp.sum(h * h, axis=1, keepdims=True))
    out[...] = h / jnp.maximum(nrm, 1e-12)


def _dense2_body(aggp, cntp, x, wl, wr, b, wfc, bfc, out):
    agg = aggp[0] + aggp[1]
    cnt = jnp.sum(cntp[...], axis=1, keepdims=True)
    mean = agg / jnp.maximum(cnt, 1.0)
    t = (jnp.dot(mean, wl[...], preferred_element_type=jnp.float32,
                 precision=lax.Precision.HIGHEST)
         + jnp.dot(x[...], wr[...], preferred_element_type=jnp.float32,
                   precision=lax.Precision.HIGHEST)
         + b[...])
    h = jnp.maximum(t, 0.0)
    nrm = jnp.sqrt(jnp.sum(h * h, axis=1, keepdims=True))
    hn = h / jnp.maximum(nrm, 1e-12)
    out[...] = jnp.dot(hn, wfc[...], preferred_element_type=jnp.float32,
                       precision=lax.Precision.HIGHEST) + bfc[...]


def _dense_common_specs():
    return [
        pl.BlockSpec((NC, BLK, F), lambda i: (0, i, 0)),
        pl.BlockSpec((BLK, NW), lambda i: (i, 0)),
        pl.BlockSpec((BLK, F), lambda i: (i, 0)),
        pl.BlockSpec((F, F), lambda i: (0, 0)),
        pl.BlockSpec((F, F), lambda i: (0, 0)),
        pl.BlockSpec((1, F), lambda i: (0, 0)),
    ]


def _dense1(aggp, cntp, x, wlT, wrT, b):
    return pl.pallas_call(
        _dense1_body,
        grid=(GRID,),
        in_specs=_dense_common_specs(),
        out_specs=pl.BlockSpec((BLK, F), lambda i: (i, 0)),
        out_shape=jax.ShapeDtypeStruct((N, F), jnp.float32),
    )(aggp, cntp, x, wlT, wrT, b)


def _dense2(aggp, cntp, x, wlT, wrT, b, wfcT, bfc):
    specs = _dense_common_specs() + [
        pl.BlockSpec((F, NCLASS), lambda i: (0, 0)),
        pl.BlockSpec((1, NCLASS), lambda i: (0, 0)),
    ]
    return pl.pallas_call(
        _dense2_body,
        grid=(GRID,),
        in_specs=specs,
        out_specs=pl.BlockSpec((BLK, NCLASS), lambda i: (i, 0)),
        out_shape=jax.ShapeDtypeStruct((N, NCLASS), jnp.float32),
    )(aggp, cntp, x, wlT, wrT, b, wfcT, bfc)


def kernel(x, edge_index, W1l, b1l, W1r, W2l, b2l, W2r, Wfc, bfc):
    ei = edge_index.astype(jnp.int32)
    pad = E_PAD - E
    ar = jnp.arange(pad, dtype=jnp.int32)
    src = jnp.concatenate([ei[0], ar % N])
    dst = jnp.concatenate([ei[1], N + ar % (NTAB - N)])

    zfeat = jnp.zeros((SCH, F), jnp.float32)
    zcnt = jnp.zeros((NTAB,), jnp.float32)

    aggp1, cntp = _make_sc_agg1()(src, dst, x, zfeat, zcnt)
    aggp1 = aggp1.reshape(NC, NTAB, F)
    cntp = cntp.reshape(NW, NTAB).T
    h = _dense1(aggp1, cntp, x, W1l.T, W1r.T, b1l.reshape(1, F))
    aggp2 = _make_sc_agg2()(src, dst, h, zfeat).reshape(NC, NTAB, F)
    out = _dense2(aggp2, cntp, h, W2l.T, W2r.T, b2l.reshape(1, F),
                  Wfc.T, bfc.reshape(1, NCLASS))
    return out
